# baseline (speedup 1.0000x reference)
import dataclasses

import jax
import jax.numpy as jnp
from jax import lax
from jax.experimental import pallas as pl
from jax.experimental.pallas import tpu as pltpu
from jax.experimental.pallas import tpu_sc as plsc

CACHE = 1_000_000
S = 16384
NC = 2
NS = 16
NW = NC * NS
GROUPS = S // 16
SENT = 2**30

CPW = 31256
B_LAST = (NW - 1) * CPW
N_LAST = CACHE - B_LAST


def _tc_dedup(idx_ref, out_ref):
    x = idx_ref[...]
    lane = lax.broadcasted_iota(jnp.int32, (128, 128), 1) % 16
    dup = jnp.zeros((128, 128), dtype=jnp.bool_)
    for sh in range(1, 16):
        y = pltpu.roll(x, 128 - sh, 1)
        dup = dup | ((y == x) & (lane < 16 - sh))
    out_ref[...] = jnp.where(dup, SENT, x)


def _sc_scatter(idx_hbm, val_hbm, cache_hbm, out_hbm, idx_v, val_v, buf,
                sem_i, sem_v, sem_b):
    c = lax.axis_index("c")
    s = lax.axis_index("s")
    w = s * NC + c
    b = w * CPW
    n = jnp.where(w == NW - 1, N_LAST, CPW).astype(jnp.uint32)

    pltpu.async_copy(idx_hbm, idx_v, sem_i)
    pltpu.async_copy(val_hbm, val_v, sem_v)

    @pl.when(w < NW - 1)
    def _():
        pltpu.async_copy(cache_hbm.at[pl.ds(b, CPW)], buf, sem_b)

    @pl.when(w == NW - 1)
    def _():
        pltpu.async_copy(cache_hbm.at[pl.ds(B_LAST, N_LAST)],
                         buf.at[pl.ds(0, N_LAST)], sem_b)

    pltpu.make_async_copy(idx_hbm, idx_v, sem_i).wait()
    pltpu.make_async_copy(val_hbm, val_v, sem_v).wait()

    @pl.when(w < NW - 1)
    def _():
        pltpu.make_async_copy(cache_hbm.at[pl.ds(b, CPW)], buf, sem_b).wait()

    @pl.when(w == NW - 1)
    def _():
        pltpu.make_async_copy(cache_hbm.at[pl.ds(B_LAST, N_LAST)],
                              buf.at[pl.ds(0, N_LAST)], sem_b).wait()

    for g in range(GROUPS):
        sl = pl.ds(g * 16, 16)
        loc = idx_v[sl] - b
        inr = plsc.bitcast(loc, jnp.uint32) < n
        plsc.store_scatter(buf, [loc], val_v[sl], mask=inr)

    @pl.when(w < NW - 1)
    def _():
        pltpu.sync_copy(buf, out_hbm.at[pl.ds(b, CPW)])

    @pl.when(w == NW - 1)
    def _():
        pltpu.sync_copy(buf.at[pl.ds(0, N_LAST)],
                        out_hbm.at[pl.ds(B_LAST, N_LAST)])


def kernel(input_pos, values, cache):
    idx2 = input_pos.astype(jnp.int32).reshape(128, 128)

    idx_d = pl.pallas_call(
        _tc_dedup,
        out_shape=jax.ShapeDtypeStruct((128, 128), jnp.int32),
    )(idx2)
    idx_flat = idx_d.reshape(S)

    mesh = plsc.VectorSubcoreMesh(core_axis_name="c", subcore_axis_name="s",
                                  num_cores=NC, num_subcores=NS)
    cp = pltpu.CompilerParams()
    if "needs_layout_passes" in pltpu.CompilerParams.__dataclass_fields__:
        cp = dataclasses.replace(cp, needs_layout_passes=False)
    run = pl.kernel(
        _sc_scatter,
        out_type=jax.ShapeDtypeStruct((CACHE,), jnp.float32),
        mesh=mesh,
        scratch_types=[
            pltpu.VMEM((S,), jnp.int32),
            pltpu.VMEM((S,), jnp.float32),
            pltpu.VMEM((CPW,), jnp.float32),
            pltpu.SemaphoreType.DMA,
            pltpu.SemaphoreType.DMA,
            pltpu.SemaphoreType.DMA,
        ],
        compiler_params=cp,
    )
    return run(idx_flat, values, cache)


# speedup vs baseline: 1.5088x; 1.5088x over previous
import dataclasses

import jax
import jax.numpy as jnp
from jax import lax
from jax.experimental import pallas as pl
from jax.experimental.pallas import tpu as pltpu
from jax.experimental.pallas import tpu_sc as plsc

CACHE = 1_000_000
S = 16384
NC = 2
NS = 16
NW = NC * NS
GROUPS = S // 16
SENT = 2**30

CPW = 31256
B_LAST = (NW - 1) * CPW
N_LAST = CACHE - B_LAST


def _tc_dedup(idx_ref, out_ref):
    x = idx_ref[...]
    lane = lax.broadcasted_iota(jnp.int32, (128, 128), 1) % 16
    dup = jnp.zeros((128, 128), dtype=jnp.bool_)
    for sh in range(1, 16):
        y = pltpu.roll(x, 128 - sh, 1)
        dup = dup | ((y == x) & (lane < 16 - sh))
    out_ref[...] = jnp.where(dup, SENT, x)


def _sc_scatter(idx_hbm, val_hbm, cache_hbm, out_hbm, idx_v, val_v, buf,
                sem_i, sem_v, sem_b):
    c = lax.axis_index("c")
    s = lax.axis_index("s")
    w = s * NC + c
    b = w * CPW
    n = jnp.where(w == NW - 1, N_LAST, CPW).astype(jnp.uint32)

    pltpu.async_copy(idx_hbm, idx_v, sem_i)
    pltpu.async_copy(val_hbm, val_v, sem_v)

    @pl.when(w < NW - 1)
    def _():
        pltpu.async_copy(cache_hbm.at[pl.ds(b, CPW)], buf, sem_b)

    @pl.when(w == NW - 1)
    def _():
        pltpu.async_copy(cache_hbm.at[pl.ds(B_LAST, N_LAST)],
                         buf.at[pl.ds(0, N_LAST)], sem_b)

    pltpu.make_async_copy(idx_hbm, idx_v, sem_i).wait()
    pltpu.make_async_copy(val_hbm, val_v, sem_v).wait()

    @pl.when(w < NW - 1)
    def _():
        pltpu.make_async_copy(cache_hbm.at[pl.ds(b, CPW)], buf, sem_b).wait()

    @pl.when(w == NW - 1)
    def _():
        pltpu.make_async_copy(cache_hbm.at[pl.ds(B_LAST, N_LAST)],
                              buf.at[pl.ds(0, N_LAST)], sem_b).wait()

    with jax.named_scope("scan"):
        @pl.loop(0, GROUPS, step=8)
        def _(g0):
            locs = []
            for t in range(8):
                sl = pl.ds((g0 + t) * 16, 16)
                loc = idx_v[sl] - b
                locs.append((sl, loc, plsc.bitcast(loc, jnp.uint32) < n))
            for sl, loc, inr in locs:
                plsc.store_scatter(buf, [loc], val_v[sl], mask=inr)

    @pl.when(w < NW - 1)
    def _():
        pltpu.sync_copy(buf, out_hbm.at[pl.ds(b, CPW)])

    @pl.when(w == NW - 1)
    def _():
        pltpu.sync_copy(buf.at[pl.ds(0, N_LAST)],
                        out_hbm.at[pl.ds(B_LAST, N_LAST)])


def kernel(input_pos, values, cache):
    idx2 = input_pos.astype(jnp.int32).reshape(128, 128)

    idx_d = pl.pallas_call(
        _tc_dedup,
        out_shape=jax.ShapeDtypeStruct((128, 128), jnp.int32),
    )(idx2)
    idx_flat = idx_d.reshape(S)

    mesh = plsc.VectorSubcoreMesh(core_axis_name="c", subcore_axis_name="s",
                                  num_cores=NC, num_subcores=NS)
    cp = pltpu.CompilerParams()
    if "needs_layout_passes" in pltpu.CompilerParams.__dataclass_fields__:
        cp = dataclasses.replace(cp, needs_layout_passes=False)
    run = pl.kernel(
        _sc_scatter,
        out_type=jax.ShapeDtypeStruct((CACHE,), jnp.float32),
        mesh=mesh,
        scratch_types=[
            pltpu.VMEM((S,), jnp.int32),
            pltpu.VMEM((S,), jnp.float32),
            pltpu.VMEM((CPW,), jnp.float32),
            pltpu.SemaphoreType.DMA,
            pltpu.SemaphoreType.DMA,
            pltpu.SemaphoreType.DMA,
        ],
        compiler_params=cp,
    )
    return run(idx_flat, values, cache)


# speedup vs baseline: 1.6823x; 1.1150x over previous
import dataclasses

import jax
import jax.numpy as jnp
from jax import lax
from jax.experimental import pallas as pl
from jax.experimental.pallas import tpu as pltpu
from jax.experimental.pallas import tpu_sc as plsc

CACHE = 1_000_000
S = 16384
NC = 2
NS = 16
NW = NC * NS
GROUPS = S // 16
SENT = 2**30

CPW = 31256
B_LAST = (NW - 1) * CPW
N_LAST = CACHE - B_LAST


def _tc_dedup(idx_ref, out_ref):
    x = idx_ref[...]
    lane = lax.broadcasted_iota(jnp.int32, (128, 128), 1) % 16
    dup = jnp.zeros((128, 128), dtype=jnp.bool_)
    for sh in range(1, 16):
        y = pltpu.roll(x, 128 - sh, 1)
        dup = dup | ((y == x) & (lane < 16 - sh))
    out_ref[...] = jnp.where(dup, SENT, x)


def _sc_scatter(idx_hbm, val_hbm, cache_hbm, out_hbm, idx_v, val_v, buf,
                idx_sh, val_sh, sem_i, sem_v, sem_b):
    c = lax.axis_index("c")
    s = lax.axis_index("s")
    w = s * NC + c
    b = w * CPW
    n = jnp.where(w == NW - 1, N_LAST, CPW).astype(jnp.uint32)

    @pl.when(w < NW - 1)
    def _():
        pltpu.async_copy(cache_hbm.at[pl.ds(b, CPW)], buf, sem_b)

    @pl.when(w == NW - 1)
    def _():
        pltpu.async_copy(cache_hbm.at[pl.ds(B_LAST, N_LAST)],
                         buf.at[pl.ds(0, N_LAST)], sem_b)

    part = pl.ds(s * (S // NS), S // NS)
    pltpu.async_copy(idx_hbm.at[part], idx_sh.at[part], sem_i)
    pltpu.async_copy(val_hbm.at[part], val_sh.at[part], sem_v)
    pltpu.make_async_copy(idx_hbm.at[part], idx_sh.at[part], sem_i).wait()
    pltpu.make_async_copy(val_hbm.at[part], val_sh.at[part], sem_v).wait()
    plsc.subcore_barrier()

    pltpu.async_copy(idx_sh, idx_v, sem_i)
    pltpu.async_copy(val_sh, val_v, sem_v)
    pltpu.make_async_copy(idx_sh, idx_v, sem_i).wait()
    pltpu.make_async_copy(val_sh, val_v, sem_v).wait()

    @pl.when(w < NW - 1)
    def _():
        pltpu.make_async_copy(cache_hbm.at[pl.ds(b, CPW)], buf, sem_b).wait()

    @pl.when(w == NW - 1)
    def _():
        pltpu.make_async_copy(cache_hbm.at[pl.ds(B_LAST, N_LAST)],
                              buf.at[pl.ds(0, N_LAST)], sem_b).wait()

    with jax.named_scope("scan"):
        @pl.loop(0, GROUPS, step=8)
        def _(g0):
            locs = []
            for t in range(8):
                sl = pl.ds((g0 + t) * 16, 16)
                loc = idx_v[sl] - b
                locs.append((sl, loc, plsc.bitcast(loc, jnp.uint32) < n))
            for sl, loc, inr in locs:
                plsc.store_scatter(buf, [loc], val_v[sl], mask=inr)

    @pl.when(w < NW - 1)
    def _():
        pltpu.sync_copy(buf, out_hbm.at[pl.ds(b, CPW)])

    @pl.when(w == NW - 1)
    def _():
        pltpu.sync_copy(buf.at[pl.ds(0, N_LAST)],
                        out_hbm.at[pl.ds(B_LAST, N_LAST)])


def kernel(input_pos, values, cache):
    idx2 = input_pos.astype(jnp.int32).reshape(128, 128)

    idx_d = pl.pallas_call(
        _tc_dedup,
        out_shape=jax.ShapeDtypeStruct((128, 128), jnp.int32),
    )(idx2)
    idx_flat = idx_d.reshape(S)

    mesh = plsc.VectorSubcoreMesh(core_axis_name="c", subcore_axis_name="s",
                                  num_cores=NC, num_subcores=NS)
    cp = pltpu.CompilerParams()
    if "needs_layout_passes" in pltpu.CompilerParams.__dataclass_fields__:
        cp = dataclasses.replace(cp, needs_layout_passes=False)
    run = pl.kernel(
        _sc_scatter,
        out_type=jax.ShapeDtypeStruct((CACHE,), jnp.float32),
        mesh=mesh,
        scratch_types=[
            pltpu.VMEM((S,), jnp.int32),
            pltpu.VMEM((S,), jnp.float32),
            pltpu.VMEM((CPW,), jnp.float32),
            pltpu.VMEM_SHARED((S,), jnp.int32),
            pltpu.VMEM_SHARED((S,), jnp.float32),
            pltpu.SemaphoreType.DMA,
            pltpu.SemaphoreType.DMA,
            pltpu.SemaphoreType.DMA,
        ],
        compiler_params=cp,
    )
    return run(idx_flat, values, cache)


# speedup vs baseline: 1.6885x; 1.0037x over previous
import dataclasses

import jax
import jax.numpy as jnp
from jax import lax
from jax.experimental import pallas as pl
from jax.experimental.pallas import tpu as pltpu
from jax.experimental.pallas import tpu_sc as plsc

CACHE = 1_000_000
S = 16384
NC = 2
NS = 16
NW = NC * NS
GROUPS = S // 16
SENT = 2**30

CPW = 31256
B_LAST = (NW - 1) * CPW
N_LAST = CACHE - B_LAST


def _tc_dedup(idx_ref, out_ref):
    x = idx_ref[...]
    lane = lax.broadcasted_iota(jnp.int32, (128, 128), 1) % 16
    dup = jnp.zeros((128, 128), dtype=jnp.bool_)
    for sh in range(1, 16):
        y = pltpu.roll(x, 128 - sh, 1)
        dup = dup | ((y == x) & (lane < 16 - sh))
    out_ref[...] = jnp.where(dup, SENT, x)


def _sc_scatter(idx_hbm, val_hbm, cache_hbm, out_hbm, idx_v, val_v, buf,
                idx_sh, val_sh, sem_i, sem_v, sem_b):
    c = lax.axis_index("c")
    s = lax.axis_index("s")
    w = s * NC + c
    b = w * CPW
    n = jnp.where(w == NW - 1, N_LAST, CPW).astype(jnp.uint32)

    @pl.when(w < NW - 1)
    def _():
        pltpu.async_copy(cache_hbm.at[pl.ds(b, CPW)], buf, sem_b)

    @pl.when(w == NW - 1)
    def _():
        pltpu.async_copy(cache_hbm.at[pl.ds(B_LAST, N_LAST)],
                         buf.at[pl.ds(0, N_LAST)], sem_b)

    part = pl.ds(s * (S // NS), S // NS)
    pltpu.async_copy(idx_hbm.at[part], idx_sh.at[part], sem_i)
    pltpu.async_copy(val_hbm.at[part], val_sh.at[part], sem_v)
    pltpu.make_async_copy(idx_hbm.at[part], idx_sh.at[part], sem_i).wait()
    pltpu.make_async_copy(val_hbm.at[part], val_sh.at[part], sem_v).wait()
    plsc.subcore_barrier()

    pltpu.async_copy(idx_sh, idx_v, sem_i)
    pltpu.async_copy(val_sh, val_v, sem_v)
    pltpu.make_async_copy(idx_sh, idx_v, sem_i).wait()
    pltpu.make_async_copy(val_sh, val_v, sem_v).wait()

    @pl.when(w < NW - 1)
    def _():
        pltpu.make_async_copy(cache_hbm.at[pl.ds(b, CPW)], buf, sem_b).wait()

    @pl.when(w == NW - 1)
    def _():
        pltpu.make_async_copy(cache_hbm.at[pl.ds(B_LAST, N_LAST)],
                              buf.at[pl.ds(0, N_LAST)], sem_b).wait()

    @pl.loop(0, GROUPS, step=16)
    def _(g0):
        locs = []
        for t in range(16):
            sl = pl.ds((g0 + t) * 16, 16)
            loc = idx_v[sl] - b
            locs.append((sl, loc, plsc.bitcast(loc, jnp.uint32) < n))
        for sl, loc, inr in locs:
            plsc.store_scatter(buf, [loc], val_v[sl], mask=inr)

    @pl.when(w < NW - 1)
    def _():
        pltpu.sync_copy(buf, out_hbm.at[pl.ds(b, CPW)])

    @pl.when(w == NW - 1)
    def _():
        pltpu.sync_copy(buf.at[pl.ds(0, N_LAST)],
                        out_hbm.at[pl.ds(B_LAST, N_LAST)])


def kernel(input_pos, values, cache):
    idx2 = input_pos.astype(jnp.int32).reshape(128, 128)

    idx_d = pl.pallas_call(
        _tc_dedup,
        out_shape=jax.ShapeDtypeStruct((128, 128), jnp.int32),
    )(idx2)
    idx_flat = idx_d.reshape(S)

    mesh = plsc.VectorSubcoreMesh(core_axis_name="c", subcore_axis_name="s",
                                  num_cores=NC, num_subcores=NS)
    cp = pltpu.CompilerParams()
    if "needs_layout_passes" in pltpu.CompilerParams.__dataclass_fields__:
        cp = dataclasses.replace(cp, needs_layout_passes=False)
    run = pl.kernel(
        _sc_scatter,
        out_type=jax.ShapeDtypeStruct((CACHE,), jnp.float32),
        mesh=mesh,
        scratch_types=[
            pltpu.VMEM((S,), jnp.int32),
            pltpu.VMEM((S,), jnp.float32),
            pltpu.VMEM((CPW,), jnp.float32),
            pltpu.VMEM_SHARED((S,), jnp.int32),
            pltpu.VMEM_SHARED((S,), jnp.float32),
            pltpu.SemaphoreType.DMA,
            pltpu.SemaphoreType.DMA,
            pltpu.SemaphoreType.DMA,
        ],
        compiler_params=cp,
    )
    return run(idx_flat, values, cache)
